# baseline (speedup 1.0000x reference)
import functools

import jax
import jax.numpy as jnp
from jax import lax
from jax.experimental import pallas as pl
from jax.experimental.pallas import tpu as pltpu
from jax.experimental.pallas import tpu_sc as plsc

B, L, H = 1024, 128, 128
BL = B * L
NF = 9
_GSIZES = (238, 192, 196, 144)
NG = len(_GSIZES)

NC, NS = 2, 16
NW = NC * NS
ROWS_PER_W = BL // NW
C = 128
NCHUNK = ROWS_PER_W // C
HV = H // 16


def _make_sc_call():
    mesh = plsc.VectorSubcoreMesh(
        core_axis_name="c", subcore_axis_name="s", num_cores=NC, num_subcores=NS
    )
    scratch = (
        [pltpu.VMEM((n * H,), jnp.float32) for n in _GSIZES]
        + [pltpu.VMEM((3 * H,), jnp.float32)]
        + [pltpu.VMEM((NF * C,), jnp.int32)]
        + [pltpu.VMEM((C * 3,), jnp.float32)]
        + [pltpu.VMEM((C * H,), jnp.float32)]
    )

    @functools.partial(
        pl.kernel,
        mesh=mesh,
        out_type=jax.ShapeDtypeStruct((BL * H,), jnp.float32),
        scratch_types=scratch,
    )
    def sc_kernel(idx_hbm, tab0, tab1, tab2, tab3, pos_hbm, wp_hbm, out_hbm,
                  tv0, tv1, tv2, tv3, wp_v, idx_v, pos_v, out_v):
        tab_hbm = (tab0, tab1, tab2, tab3)
        tab_v = (tv0, tv1, tv2, tv3)

        wid = lax.axis_index("s") * NC + lax.axis_index("c")
        base0 = wid * ROWS_PER_W

        for g in range(NG):
            pltpu.sync_copy(tab_hbm[g], tab_v[g])
        pltpu.sync_copy(wp_hbm, wp_v)

        def chunk_body(it, carry_outer):
            base = base0 + it * C
            pltpu.sync_copy(idx_hbm.at[pl.ds(base * NF, NF * C)], idx_v)
            pltpu.sync_copy(pos_hbm.at[pl.ds(base * 3, C * 3)], pos_v)

            wp_vecs = tuple(
                wp_v[pl.ds(k * H + j * 16, 16)] for k in range(3) for j in range(HV)
            )

            @plsc.parallel_loop(0, C // 16, unroll=2, carry=wp_vecs)
            def group_body(g, wp_c):
                iv = [idx_v[pl.ds((g * 16 * NF) + f * 16, 16)]
                      for f in range(NF)]
                cv = [
                    iv[0] * 2 + iv[5],
                    iv[1] * 12 + iv[2],
                    iv[3] * 14 + iv[4],
                    (iv[6] * 9 + iv[7]) * 2 + iv[8],
                ]
                pvecs = [pos_v[pl.ds(g * 48 + m * 16, 16)] for m in range(3)]
                for rr in range(16):
                    idx = [cv[t][rr] for t in range(NG)]
                    pv = [
                        jnp.full((16,),
                                 pvecs[(rr * 3 + k) // 16][(rr * 3 + k) % 16],
                                 jnp.float32)
                        for k in range(3)
                    ]
                    rowoff = (g * 16 + rr) * H
                    for j in range(HV):
                        t01 = (tab_v[0][pl.ds(idx[0] * H + j * 16, 16)]
                               + tab_v[1][pl.ds(idx[1] * H + j * 16, 16)])
                        t23 = (tab_v[2][pl.ds(idx[2] * H + j * 16, 16)]
                               + tab_v[3][pl.ds(idx[3] * H + j * 16, 16)])
                        pacc = (pv[0] * wp_c[j] + pv[1] * wp_c[HV + j]
                                + pv[2] * wp_c[2 * HV + j])
                        out_v[pl.ds(rowoff + j * 16, 16)] = (t01 + t23) + pacc
                return wp_c

            pltpu.sync_copy(out_v, out_hbm.at[pl.ds(base * H, C * H)])
            return carry_outer

        lax.fori_loop(0, NCHUNK, chunk_body, 0)

    return sc_kernel


_SC_CALL = _make_sc_call()


def kernel(atomic_number, formal_charge, degree, explicit_valence,
           implicit_valence, aromatic, hybridization, total_num_H, is_in_ring,
           W_atomic_number, W_formal_charge, W_degree, W_explicit_valence,
           W_implicit_valence, W_aromatic, W_hybridization, W_total_num_H,
           W_is_in_ring, position, W_position):
    idxs = [atomic_number, formal_charge, degree, explicit_valence,
            implicit_valence, aromatic, hybridization, total_num_H, is_in_ring]
    idx = jnp.stack([i.reshape(BL).astype(jnp.int32) for i in idxs])
    idx = (idx.reshape(NF, NW * NCHUNK * (C // 16), 16)
           .transpose(1, 0, 2).reshape(-1))
    f32 = jnp.float32
    g0 = (W_atomic_number.astype(f32)[:, None, :]
          + W_aromatic.astype(f32)[None, :, :]).reshape(-1)
    g1 = (W_formal_charge.astype(f32)[:, None, :]
          + W_degree.astype(f32)[None, :, :]).reshape(-1)
    g2 = (W_explicit_valence.astype(f32)[:, None, :]
          + W_implicit_valence.astype(f32)[None, :, :]).reshape(-1)
    g3 = (W_hybridization.astype(f32)[:, None, None, :]
          + W_total_num_H.astype(f32)[None, :, None, :]
          + W_is_in_ring.astype(f32)[None, None, :, :]).reshape(-1)
    pos = position.reshape(BL * 3).astype(f32)
    wp = W_position.reshape(3 * H).astype(f32)
    out = _SC_CALL(idx, g0, g1, g2, g3, pos, wp)
    return out.reshape(B, L, H)


# speedup vs baseline: 1.8932x; 1.8932x over previous
import functools

import jax
import jax.numpy as jnp
from jax import lax
from jax.experimental import pallas as pl
from jax.experimental.pallas import tpu as pltpu
from jax.experimental.pallas import tpu_sc as plsc

B, L, H = 1024, 128, 128
BL = B * L
NF = 9
_GSIZES = (238, 192, 196, 144)
NG = len(_GSIZES)

NC, NS = 2, 16
NW = NC * NS
ROWS_PER_W = BL // NW
C = 128
NCHUNK = ROWS_PER_W // C
HV = H // 16


def _make_sc_call():
    mesh = plsc.VectorSubcoreMesh(
        core_axis_name="c", subcore_axis_name="s", num_cores=NC, num_subcores=NS
    )
    scratch = (
        [pltpu.VMEM((n * H,), jnp.float32) for n in _GSIZES]
        + [pltpu.VMEM((3 * H,), jnp.float32)]
        + [pltpu.VMEM((NF * C,), jnp.int32)]
        + [pltpu.VMEM((C * 3,), jnp.float32)]
        + [pltpu.VMEM((C * H,), jnp.float32)]
    )

    @functools.partial(
        pl.kernel,
        mesh=mesh,
        out_type=jax.ShapeDtypeStruct((BL * H,), jnp.float32),
        scratch_types=scratch,
    )
    def sc_kernel(idx_hbm, tab0, tab1, tab2, tab3, pos_hbm, wp_hbm, out_hbm,
                  tv0, tv1, tv2, tv3, wp_v, idx_v, pos_v, out_v):
        tab_hbm = (tab0, tab1, tab2, tab3)
        tab_v = (tv0, tv1, tv2, tv3)

        wid = lax.axis_index("s") * NC + lax.axis_index("c")
        base0 = wid * ROWS_PER_W

        for g in range(NG):
            pltpu.sync_copy(tab_hbm[g], tab_v[g])
        pltpu.sync_copy(wp_hbm, wp_v)

        def chunk_body(it, carry_outer):
            base = base0 + it * C
            pltpu.sync_copy(idx_hbm.at[pl.ds(base * NF, NF * C)], idx_v)
            pltpu.sync_copy(pos_hbm.at[pl.ds(base * 3, C * 3)], pos_v)

            wp_vecs = tuple(
                wp_v[pl.ds(k * H + j * 16, 16)] for k in range(3) for j in range(HV)
            )

            @plsc.parallel_loop(0, C // 16, carry=wp_vecs)
            def group_body(g, wp_c):
                iv = [idx_v[pl.ds((g * 16 * NF) + f * 16, 16)]
                      for f in range(NF)]
                cv = [
                    iv[0] * 2 + iv[5],
                    iv[1] * 12 + iv[2],
                    iv[3] * 14 + iv[4],
                    (iv[6] * 9 + iv[7]) * 2 + iv[8],
                ]
                pvecs = [pos_v[pl.ds(g * 48 + m * 16, 16)] for m in range(3)]

                def extracts(rr):
                    idx = [cv[t][rr] for t in range(NG)]
                    pv = [
                        jnp.full((16,),
                                 pvecs[(rr * 3 + k) // 16][(rr * 3 + k) % 16],
                                 jnp.float32)
                        for k in range(3)
                    ]
                    return idx, pv

                def compute(pv, ld, j, rowoff):
                    pacc = (pv[0] * wp_c[j] + pv[1] * wp_c[HV + j]
                            + pv[2] * wp_c[2 * HV + j])
                    out_v[pl.ds(rowoff + j * 16, 16)] = (
                        (ld[0] + ld[1]) + (ld[2] + ld[3])) + pacc

                cur = extracts(0)
                pending = None
                for rr in range(16):
                    idx_cur, pv_cur = cur
                    nxt = extracts(rr + 1) if rr < 15 else None
                    rowoff = (g * 16 + rr) * H
                    for j in range(HV):
                        ld = [tab_v[t][pl.ds(idx_cur[t] * H + j * 16, 16)]
                              for t in range(NG)]
                        if pending is not None:
                            compute(*pending)
                        pending = (pv_cur, ld, j, rowoff)
                    cur = nxt
                compute(*pending)
                return wp_c

            pltpu.sync_copy(out_v, out_hbm.at[pl.ds(base * H, C * H)])
            return carry_outer

        lax.fori_loop(0, NCHUNK, chunk_body, 0)

    return sc_kernel


_SC_CALL = _make_sc_call()


def kernel(atomic_number, formal_charge, degree, explicit_valence,
           implicit_valence, aromatic, hybridization, total_num_H, is_in_ring,
           W_atomic_number, W_formal_charge, W_degree, W_explicit_valence,
           W_implicit_valence, W_aromatic, W_hybridization, W_total_num_H,
           W_is_in_ring, position, W_position):
    idxs = [atomic_number, formal_charge, degree, explicit_valence,
            implicit_valence, aromatic, hybridization, total_num_H, is_in_ring]
    idx = jnp.stack([i.reshape(BL).astype(jnp.int32) for i in idxs])
    idx = (idx.reshape(NF, NW * NCHUNK * (C // 16), 16)
           .transpose(1, 0, 2).reshape(-1))
    f32 = jnp.float32
    g0 = (W_atomic_number.astype(f32)[:, None, :]
          + W_aromatic.astype(f32)[None, :, :]).reshape(-1)
    g1 = (W_formal_charge.astype(f32)[:, None, :]
          + W_degree.astype(f32)[None, :, :]).reshape(-1)
    g2 = (W_explicit_valence.astype(f32)[:, None, :]
          + W_implicit_valence.astype(f32)[None, :, :]).reshape(-1)
    g3 = (W_hybridization.astype(f32)[:, None, None, :]
          + W_total_num_H.astype(f32)[None, :, None, :]
          + W_is_in_ring.astype(f32)[None, None, :, :]).reshape(-1)
    pos = position.reshape(BL * 3).astype(f32)
    wp = W_position.reshape(3 * H).astype(f32)
    out = _SC_CALL(idx, g0, g1, g2, g3, pos, wp)
    return out.reshape(B, L, H)


# speedup vs baseline: 2.1141x; 1.1166x over previous
import functools

import jax
import jax.numpy as jnp
from jax import lax
from jax.experimental import pallas as pl
from jax.experimental.pallas import tpu as pltpu
from jax.experimental.pallas import tpu_sc as plsc

B, L, H = 1024, 128, 128
BL = B * L
NF = 9
_GSIZES = (238, 192, 196, 144)
NG = len(_GSIZES)

NC, NS = 2, 16
NW = NC * NS
ROWS_PER_W = BL // NW
C = 128
NCHUNK = ROWS_PER_W // C
HV = H // 16


def _make_sc_call():
    mesh = plsc.VectorSubcoreMesh(
        core_axis_name="c", subcore_axis_name="s", num_cores=NC, num_subcores=NS
    )
    scratch = (
        [pltpu.VMEM((n * H,), jnp.float32) for n in _GSIZES]
        + [pltpu.VMEM((3 * H,), jnp.float32)]
        + [pltpu.VMEM((NF * C,), jnp.int32)]
        + [pltpu.VMEM((C * 3,), jnp.float32)]
        + [pltpu.VMEM((C * H,), jnp.float32)]
    )

    @functools.partial(
        pl.kernel,
        mesh=mesh,
        out_type=jax.ShapeDtypeStruct((BL * H,), jnp.float32),
        scratch_types=scratch,
    )
    def sc_kernel(idx_hbm, tab0, tab1, tab2, tab3, pos_hbm, wp_hbm, out_hbm,
                  tv0, tv1, tv2, tv3, wp_v, idx_v, pos_v, out_v):
        tab_hbm = (tab0, tab1, tab2, tab3)
        tab_v = (tv0, tv1, tv2, tv3)

        wid = lax.axis_index("s") * NC + lax.axis_index("c")
        base0 = wid * ROWS_PER_W

        for g in range(NG):
            pltpu.sync_copy(tab_hbm[g], tab_v[g])
        pltpu.sync_copy(wp_hbm, wp_v)

        def chunk_body(it, carry_outer):
            base = base0 + it * C
            pltpu.sync_copy(idx_hbm.at[pl.ds(base * NF, NF * C)], idx_v)
            pltpu.sync_copy(pos_hbm.at[pl.ds(base * 3, C * 3)], pos_v)

            wp_vecs = tuple(
                wp_v[pl.ds(k * H + j * 16, 16)] for k in range(3) for j in range(HV)
            )

            @plsc.parallel_loop(0, C // 16, carry=wp_vecs)
            def group_body(g, wp_c):
                iv = [idx_v[pl.ds((g * 16 * NF) + f * 16, 16)]
                      for f in range(NF)]
                cv = [
                    iv[0] * 2 + iv[5],
                    iv[1] * 12 + iv[2],
                    iv[3] * 14 + iv[4],
                    (iv[6] * 9 + iv[7]) * 2 + iv[8],
                ]
                pvecs = [pos_v[pl.ds(g * 48 + m * 16, 16)] for m in range(3)]

                def extracts(rr):
                    idx = [cv[t][rr] for t in range(NG)]
                    pv = [
                        jnp.full((16,),
                                 pvecs[(rr * 3 + k) // 16][(rr * 3 + k) % 16],
                                 jnp.float32)
                        for k in range(3)
                    ]
                    return idx, pv

                def compute(pv, ld, j, rowoff):
                    pacc = (pv[0] * wp_c[j] + pv[1] * wp_c[HV + j]
                            + pv[2] * wp_c[2 * HV + j])
                    out_v[pl.ds(rowoff + j * 16, 16)] = (
                        (ld[0] + ld[1]) + (ld[2] + ld[3])) + pacc

                DEPTH = 2
                cur = extracts(0)
                pending = []
                for rr in range(16):
                    idx_cur, pv_cur = cur
                    nxt = extracts(rr + 1) if rr < 15 else None
                    rowoff = (g * 16 + rr) * H
                    for j in range(HV):
                        ld = [tab_v[t][pl.ds(idx_cur[t] * H + j * 16, 16)]
                              for t in range(NG)]
                        pending.append((pv_cur, ld, j, rowoff))
                        if len(pending) > DEPTH:
                            compute(*pending.pop(0))
                    cur = nxt
                for p in pending:
                    compute(*p)
                return wp_c

            pltpu.sync_copy(out_v, out_hbm.at[pl.ds(base * H, C * H)])
            return carry_outer

        lax.fori_loop(0, NCHUNK, chunk_body, 0)

    return sc_kernel


_SC_CALL = _make_sc_call()


def kernel(atomic_number, formal_charge, degree, explicit_valence,
           implicit_valence, aromatic, hybridization, total_num_H, is_in_ring,
           W_atomic_number, W_formal_charge, W_degree, W_explicit_valence,
           W_implicit_valence, W_aromatic, W_hybridization, W_total_num_H,
           W_is_in_ring, position, W_position):
    idxs = [atomic_number, formal_charge, degree, explicit_valence,
            implicit_valence, aromatic, hybridization, total_num_H, is_in_ring]
    idx = jnp.stack([i.reshape(BL).astype(jnp.int32) for i in idxs])
    idx = (idx.reshape(NF, NW * NCHUNK * (C // 16), 16)
           .transpose(1, 0, 2).reshape(-1))
    f32 = jnp.float32
    g0 = (W_atomic_number.astype(f32)[:, None, :]
          + W_aromatic.astype(f32)[None, :, :]).reshape(-1)
    g1 = (W_formal_charge.astype(f32)[:, None, :]
          + W_degree.astype(f32)[None, :, :]).reshape(-1)
    g2 = (W_explicit_valence.astype(f32)[:, None, :]
          + W_implicit_valence.astype(f32)[None, :, :]).reshape(-1)
    g3 = (W_hybridization.astype(f32)[:, None, None, :]
          + W_total_num_H.astype(f32)[None, :, None, :]
          + W_is_in_ring.astype(f32)[None, None, :, :]).reshape(-1)
    pos = position.reshape(BL * 3).astype(f32)
    wp = W_position.reshape(3 * H).astype(f32)
    out = _SC_CALL(idx, g0, g1, g2, g3, pos, wp)
    return out.reshape(B, L, H)


# speedup vs baseline: 2.2133x; 1.0469x over previous
import functools

import jax
import jax.numpy as jnp
from jax import lax
from jax.experimental import pallas as pl
from jax.experimental.pallas import tpu as pltpu
from jax.experimental.pallas import tpu_sc as plsc

B, L, H = 1024, 128, 128
BL = B * L
NF = 9
_GSIZES = (238, 192, 196, 144)
NG = len(_GSIZES)

NC, NS = 2, 16
NW = NC * NS
ROWS_PER_W = BL // NW
C = 128
NCHUNK = ROWS_PER_W // C
HV = H // 16


def _make_sc_call():
    mesh = plsc.VectorSubcoreMesh(
        core_axis_name="c", subcore_axis_name="s", num_cores=NC, num_subcores=NS
    )
    scratch = (
        [pltpu.VMEM((n * H,), jnp.float32) for n in _GSIZES]
        + [pltpu.VMEM((3 * H,), jnp.float32)]
        + [pltpu.VMEM((NF * C,), jnp.int32)]
        + [pltpu.VMEM((C * 3,), jnp.float32)]
        + [pltpu.VMEM((C * H,), jnp.float32)]
    )

    @functools.partial(
        pl.kernel,
        mesh=mesh,
        out_type=jax.ShapeDtypeStruct((BL * H,), jnp.float32),
        scratch_types=scratch,
    )
    def sc_kernel(idx_hbm, tab0, tab1, tab2, tab3, pos_hbm, wp_hbm, out_hbm,
                  tv0, tv1, tv2, tv3, wp_v, idx_v, pos_v, out_v):
        tab_hbm = (tab0, tab1, tab2, tab3)
        tab_v = (tv0, tv1, tv2, tv3)

        wid = lax.axis_index("s") * NC + lax.axis_index("c")
        base0 = wid * ROWS_PER_W

        for g in range(NG):
            pltpu.sync_copy(tab_hbm[g], tab_v[g])
        pltpu.sync_copy(wp_hbm, wp_v)

        def chunk_body(it, carry_outer):
            base = base0 + it * C
            pltpu.sync_copy(idx_hbm.at[pl.ds(base * NF, NF * C)], idx_v)
            pltpu.sync_copy(pos_hbm.at[pl.ds(base * 3, C * 3)], pos_v)

            wp_vecs = tuple(
                wp_v[pl.ds(k * H + j * 16, 16)] for k in range(3) for j in range(HV)
            )

            @plsc.parallel_loop(0, C // 16, carry=wp_vecs)
            def group_body(g, wp_c):
                iv = [idx_v[pl.ds((g * 16 * NF) + f * 16, 16)]
                      for f in range(NF)]
                cv = [
                    iv[0] * 2 + iv[5],
                    iv[1] * 12 + iv[2],
                    iv[3] * 14 + iv[4],
                    (iv[6] * 9 + iv[7]) * 2 + iv[8],
                ]
                pvecs = [pos_v[pl.ds(g * 48 + m * 16, 16)] for m in range(3)]

                def extracts(rr):
                    idx = [cv[t][rr] for t in range(NG)]
                    pv = [
                        jnp.full((16,),
                                 pvecs[(rr * 3 + k) // 16][(rr * 3 + k) % 16],
                                 jnp.float32)
                        for k in range(3)
                    ]
                    return idx, pv

                def compute(pv, ld, j, rowoff):
                    pacc = (pv[0] * wp_c[j] + pv[1] * wp_c[HV + j]
                            + pv[2] * wp_c[2 * HV + j])
                    out_v[pl.ds(rowoff + j * 16, 16)] = (
                        (ld[0] + ld[1]) + (ld[2] + ld[3])) + pacc

                DEPTH = 3
                cur = extracts(0)
                pending = []
                for rr in range(16):
                    idx_cur, pv_cur = cur
                    nxt = extracts(rr + 1) if rr < 15 else None
                    rowoff = (g * 16 + rr) * H
                    for j in range(HV):
                        ld = [tab_v[t][pl.ds(idx_cur[t] * H + j * 16, 16)]
                              for t in range(NG)]
                        pending.append((pv_cur, ld, j, rowoff))
                        if len(pending) > DEPTH:
                            compute(*pending.pop(0))
                    cur = nxt
                for p in pending:
                    compute(*p)
                return wp_c

            pltpu.sync_copy(out_v, out_hbm.at[pl.ds(base * H, C * H)])
            return carry_outer

        lax.fori_loop(0, NCHUNK, chunk_body, 0)

    return sc_kernel


_SC_CALL = _make_sc_call()


def kernel(atomic_number, formal_charge, degree, explicit_valence,
           implicit_valence, aromatic, hybridization, total_num_H, is_in_ring,
           W_atomic_number, W_formal_charge, W_degree, W_explicit_valence,
           W_implicit_valence, W_aromatic, W_hybridization, W_total_num_H,
           W_is_in_ring, position, W_position):
    idxs = [atomic_number, formal_charge, degree, explicit_valence,
            implicit_valence, aromatic, hybridization, total_num_H, is_in_ring]
    idx = jnp.stack([i.reshape(BL).astype(jnp.int32) for i in idxs])
    idx = (idx.reshape(NF, NW * NCHUNK * (C // 16), 16)
           .transpose(1, 0, 2).reshape(-1))
    f32 = jnp.float32
    g0 = (W_atomic_number.astype(f32)[:, None, :]
          + W_aromatic.astype(f32)[None, :, :]).reshape(-1)
    g1 = (W_formal_charge.astype(f32)[:, None, :]
          + W_degree.astype(f32)[None, :, :]).reshape(-1)
    g2 = (W_explicit_valence.astype(f32)[:, None, :]
          + W_implicit_valence.astype(f32)[None, :, :]).reshape(-1)
    g3 = (W_hybridization.astype(f32)[:, None, None, :]
          + W_total_num_H.astype(f32)[None, :, None, :]
          + W_is_in_ring.astype(f32)[None, None, :, :]).reshape(-1)
    pos = position.reshape(BL * 3).astype(f32)
    wp = W_position.reshape(3 * H).astype(f32)
    out = _SC_CALL(idx, g0, g1, g2, g3, pos, wp)
    return out.reshape(B, L, H)


# speedup vs baseline: 2.5231x; 1.1400x over previous
import functools

import jax
import jax.numpy as jnp
from jax import lax
from jax.experimental import pallas as pl
from jax.experimental.pallas import tpu as pltpu
from jax.experimental.pallas import tpu_sc as plsc

B, L, H = 1024, 128, 128
BL = B * L
NF = 9
_GSIZES = (238, 192, 196, 144)
NG = len(_GSIZES)

NC, NS = 2, 16
NW = NC * NS
ROWS_PER_W = BL // NW
C = 64
NCHUNK = ROWS_PER_W // C
HV = H // 16
REC = (NF + 3) * C


def _make_sc_call():
    mesh = plsc.VectorSubcoreMesh(
        core_axis_name="c", subcore_axis_name="s", num_cores=NC, num_subcores=NS
    )
    scratch = (
        [pltpu.VMEM((n * H,), jnp.float32) for n in _GSIZES]
        + [pltpu.VMEM((3 * H,), jnp.float32)]
        + [pltpu.VMEM((REC,), jnp.int32) for _ in range(2)]
        + [pltpu.VMEM((C * H,), jnp.float32) for _ in range(2)]
        + [pltpu.SemaphoreType.DMA for _ in range(4)]
    )

    @functools.partial(
        pl.kernel,
        mesh=mesh,
        out_type=jax.ShapeDtypeStruct((BL * H,), jnp.float32),
        scratch_types=scratch,
    )
    def sc_kernel(rec_hbm, tab0, tab1, tab2, tab3, wp_hbm, out_hbm,
                  tv0, tv1, tv2, tv3, wp_v, rv0, rv1, ov0, ov1,
                  si0, si1, so0, so1):
        tab_v = (tv0, tv1, tv2, tv3)
        rec_v = (rv0, rv1)
        out_v = (ov0, ov1)
        sin = (si0, si1)
        sout = (so0, so1)

        wid = lax.axis_index("s") * NC + lax.axis_index("c")
        base0 = wid * ROWS_PER_W
        rec0 = wid * NCHUNK * REC

        for g, t in enumerate((tab0, tab1, tab2, tab3)):
            pltpu.sync_copy(t, tab_v[g])
        pltpu.sync_copy(wp_hbm, wp_v)

        def in_descr(chunk, slot):
            return pltpu.make_async_copy(
                rec_hbm.at[pl.ds(rec0 + chunk * REC, REC)], rec_v[slot],
                sin[slot])

        def out_descr(chunk, slot):
            base = base0 + chunk * C
            return pltpu.make_async_copy(
                out_v[slot], out_hbm.at[pl.ds(base * H, C * H)], sout[slot])

        in_descr(0, 0).start()
        in_descr(1, 1).start()

        def compute_chunk(slot):
            wp_vecs = tuple(
                wp_v[pl.ds(k * H + j * 16, 16)] for k in range(3) for j in range(HV)
            )
            rv = rec_v[slot]
            ov = out_v[slot]

            @plsc.parallel_loop(0, C // 16, carry=wp_vecs)
            def group_body(g, wp_c):
                iv = [rv[pl.ds((g * 16 * NF) + f * 16, 16)] for f in range(NF)]
                cv = [
                    iv[0] * 2 + iv[5],
                    iv[1] * 12 + iv[2],
                    iv[3] * 14 + iv[4],
                    (iv[6] * 9 + iv[7]) * 2 + iv[8],
                ]
                pvecs = [
                    lax.bitcast_convert_type(
                        rv[pl.ds(NF * C + g * 48 + m * 16, 16)], jnp.float32)
                    for m in range(3)
                ]

                def extracts(rr):
                    idx = [cv[t][rr] for t in range(NG)]
                    pv = [
                        jnp.full((16,),
                                 pvecs[(rr * 3 + k) // 16][(rr * 3 + k) % 16],
                                 jnp.float32)
                        for k in range(3)
                    ]
                    return idx, pv

                def compute(pv, ld, j, rowoff):
                    pacc = (pv[0] * wp_c[j] + pv[1] * wp_c[HV + j]
                            + pv[2] * wp_c[2 * HV + j])
                    ov[pl.ds(rowoff + j * 16, 16)] = (
                        (ld[0] + ld[1]) + (ld[2] + ld[3])) + pacc

                DEPTH = 3
                cur = extracts(0)
                pending = []
                for rr in range(16):
                    idx_cur, pv_cur = cur
                    nxt = extracts(rr + 1) if rr < 15 else None
                    rowoff = (g * 16 + rr) * H
                    for j in range(HV):
                        ld = [tab_v[t][pl.ds(idx_cur[t] * H + j * 16, 16)]
                              for t in range(NG)]
                        pending.append((pv_cur, ld, j, rowoff))
                        if len(pending) > DEPTH:
                            compute(*pending.pop(0))
                    cur = nxt
                for p in pending:
                    compute(*p)
                return wp_c

        def body2(i2, carry):
            for slot in range(2):
                chunk = i2 * 2 + slot
                in_descr(chunk, slot).wait()

                @pl.when(i2 > 0)
                def _():
                    out_descr(chunk - 2, slot).wait()

                compute_chunk(slot)
                out_descr(chunk, slot).start()

                @pl.when(chunk + 2 < NCHUNK)
                def _():
                    in_descr(chunk + 2, slot).start()

            return carry

        lax.fori_loop(0, NCHUNK // 2, body2, 0)
        out_descr(NCHUNK - 2, 0).wait()
        out_descr(NCHUNK - 1, 1).wait()

    return sc_kernel


_SC_CALL = _make_sc_call()


def kernel(atomic_number, formal_charge, degree, explicit_valence,
           implicit_valence, aromatic, hybridization, total_num_H, is_in_ring,
           W_atomic_number, W_formal_charge, W_degree, W_explicit_valence,
           W_implicit_valence, W_aromatic, W_hybridization, W_total_num_H,
           W_is_in_ring, position, W_position):
    idxs = [atomic_number, formal_charge, degree, explicit_valence,
            implicit_valence, aromatic, hybridization, total_num_H, is_in_ring]
    idx = jnp.stack([i.reshape(BL).astype(jnp.int32) for i in idxs])
    idx = (idx.reshape(NF, NW * NCHUNK * (C // 16), 16)
           .transpose(1, 0, 2).reshape(NW * NCHUNK, NF * C))
    f32 = jnp.float32
    posi = lax.bitcast_convert_type(
        position.reshape(NW * NCHUNK, C * 3).astype(f32), jnp.int32)
    rec = jnp.concatenate([idx, posi], axis=1).reshape(-1)
    g0 = (W_atomic_number.astype(f32)[:, None, :]
          + W_aromatic.astype(f32)[None, :, :]).reshape(-1)
    g1 = (W_formal_charge.astype(f32)[:, None, :]
          + W_degree.astype(f32)[None, :, :]).reshape(-1)
    g2 = (W_explicit_valence.astype(f32)[:, None, :]
          + W_implicit_valence.astype(f32)[None, :, :]).reshape(-1)
    g3 = (W_hybridization.astype(f32)[:, None, None, :]
          + W_total_num_H.astype(f32)[None, :, None, :]
          + W_is_in_ring.astype(f32)[None, None, :, :]).reshape(-1)
    wp = W_position.reshape(3 * H).astype(f32)
    out = _SC_CALL(rec, g0, g1, g2, g3, wp)
    return out.reshape(B, L, H)


# speedup vs baseline: 4.0540x; 1.6068x over previous
import functools

import jax
import jax.numpy as jnp
from jax import lax
from jax.experimental import pallas as pl
from jax.experimental.pallas import tpu as pltpu
from jax.experimental.pallas import tpu_sc as plsc

B, L, H = 1024, 128, 128
BL = B * L
NF = 9
_GSIZES = (238, 192, 196, 144)
NG = len(_GSIZES)

NC, NS = 2, 16
NW = NC * NS
ROWS_PER_W = BL // NW
C = 64
NCHUNK = ROWS_PER_W // C
HV = H // 16


def _make_sc_call():
    mesh = plsc.VectorSubcoreMesh(
        core_axis_name="c", subcore_axis_name="s", num_cores=NC, num_subcores=NS
    )
    scratch = (
        [pltpu.VMEM((n * H,), jnp.float32) for n in _GSIZES]
        + [pltpu.VMEM((3 * H,), jnp.float32)]
        + [pltpu.VMEM((NF, C), jnp.int32) for _ in range(2)]
        + [pltpu.VMEM((C * 3,), jnp.float32) for _ in range(2)]
        + [pltpu.VMEM((C * H,), jnp.float32) for _ in range(2)]
        + [pltpu.SemaphoreType.DMA for _ in range(4)]
    )

    @functools.partial(
        pl.kernel,
        mesh=mesh,
        out_type=jax.ShapeDtypeStruct((BL * H,), jnp.float32),
        scratch_types=scratch,
    )
    def sc_kernel(i0, i1, i2, i3, i4, i5, i6, i7, i8,
                  tab0, tab1, tab2, tab3, pos_hbm, wp_hbm, out_hbm,
                  tv0, tv1, tv2, tv3, wp_v, xv0, xv1, pv0, pv1, ov0, ov1,
                  si0, si1, so0, so1):
        idx_hbm = (i0, i1, i2, i3, i4, i5, i6, i7, i8)
        tab_v = (tv0, tv1, tv2, tv3)
        idx_v = (xv0, xv1)
        pos_v = (pv0, pv1)
        out_v = (ov0, ov1)
        sin = (si0, si1)
        sout = (so0, so1)

        wid = lax.axis_index("s") * NC + lax.axis_index("c")
        base0 = wid * ROWS_PER_W

        for g, t in enumerate((tab0, tab1, tab2, tab3)):
            pltpu.sync_copy(t, tab_v[g])
        pltpu.sync_copy(wp_hbm, wp_v)

        def in_descrs(chunk, slot):
            base = base0 + chunk * C
            ds = [pltpu.make_async_copy(idx_hbm[f].at[pl.ds(base, C)],
                                        idx_v[slot].at[f], sin[slot])
                  for f in range(NF)]
            ds.append(pltpu.make_async_copy(
                pos_hbm.at[pl.ds(base * 3, C * 3)], pos_v[slot], sin[slot]))
            return ds

        def out_descr(chunk, slot):
            base = base0 + chunk * C
            return pltpu.make_async_copy(
                out_v[slot], out_hbm.at[pl.ds(base * H, C * H)], sout[slot])

        for d in in_descrs(0, 0):
            d.start()
        for d in in_descrs(1, 1):
            d.start()

        def compute_chunk(slot):
            wp_vecs = tuple(
                wp_v[pl.ds(k * H + j * 16, 16)] for k in range(3) for j in range(HV)
            )
            ov = out_v[slot]
            pvr = pos_v[slot]
            xvr = idx_v[slot]

            @plsc.parallel_loop(0, C // 16, carry=wp_vecs)
            def group_body(g, wp_c):
                iv = [xvr[f, pl.ds(g * 16, 16)] for f in range(NF)]
                cv = [
                    iv[0] * 2 + iv[5],
                    iv[1] * 12 + iv[2],
                    iv[3] * 14 + iv[4],
                    (iv[6] * 9 + iv[7]) * 2 + iv[8],
                ]
                pvecs = [pvr[pl.ds(g * 48 + m * 16, 16)] for m in range(3)]

                def extracts(rr):
                    idx = [cv[t][rr] for t in range(NG)]
                    pv = [
                        jnp.full((16,),
                                 pvecs[(rr * 3 + k) // 16][(rr * 3 + k) % 16],
                                 jnp.float32)
                        for k in range(3)
                    ]
                    return idx, pv

                def compute(pv, ld, j, rowoff):
                    pacc = (pv[0] * wp_c[j] + pv[1] * wp_c[HV + j]
                            + pv[2] * wp_c[2 * HV + j])
                    ov[pl.ds(rowoff + j * 16, 16)] = (
                        (ld[0] + ld[1]) + (ld[2] + ld[3])) + pacc

                DEPTH = 3
                cur = extracts(0)
                pending = []
                for rr in range(16):
                    idx_cur, pv_cur = cur
                    nxt = extracts(rr + 1) if rr < 15 else None
                    rowoff = (g * 16 + rr) * H
                    for j in range(HV):
                        ld = [tab_v[t][pl.ds(idx_cur[t] * H + j * 16, 16)]
                              for t in range(NG)]
                        pending.append((pv_cur, ld, j, rowoff))
                        if len(pending) > DEPTH:
                            compute(*pending.pop(0))
                    cur = nxt
                for p in pending:
                    compute(*p)
                return wp_c

        def body2(i2_, carry):
            for slot in range(2):
                chunk = i2_ * 2 + slot
                for d in in_descrs(chunk, slot):
                    d.wait()

                @pl.when(i2_ > 0)
                def _():
                    out_descr(chunk - 2, slot).wait()

                compute_chunk(slot)
                out_descr(chunk, slot).start()

                @pl.when(chunk + 2 < NCHUNK)
                def _():
                    for d in in_descrs(chunk + 2, slot):
                        d.start()

            return carry

        lax.fori_loop(0, NCHUNK // 2, body2, 0)
        out_descr(NCHUNK - 2, 0).wait()
        out_descr(NCHUNK - 1, 1).wait()

    return sc_kernel


_SC_CALL = _make_sc_call()


def kernel(atomic_number, formal_charge, degree, explicit_valence,
           implicit_valence, aromatic, hybridization, total_num_H, is_in_ring,
           W_atomic_number, W_formal_charge, W_degree, W_explicit_valence,
           W_implicit_valence, W_aromatic, W_hybridization, W_total_num_H,
           W_is_in_ring, position, W_position):
    idxs = [atomic_number, formal_charge, degree, explicit_valence,
            implicit_valence, aromatic, hybridization, total_num_H, is_in_ring]
    idxs = [i.reshape(BL).astype(jnp.int32) for i in idxs]
    f32 = jnp.float32
    g0 = (W_atomic_number.astype(f32)[:, None, :]
          + W_aromatic.astype(f32)[None, :, :]).reshape(-1)
    g1 = (W_formal_charge.astype(f32)[:, None, :]
          + W_degree.astype(f32)[None, :, :]).reshape(-1)
    g2 = (W_explicit_valence.astype(f32)[:, None, :]
          + W_implicit_valence.astype(f32)[None, :, :]).reshape(-1)
    g3 = (W_hybridization.astype(f32)[:, None, None, :]
          + W_total_num_H.astype(f32)[None, :, None, :]
          + W_is_in_ring.astype(f32)[None, None, :, :]).reshape(-1)
    pos = position.reshape(BL * 3).astype(f32)
    wp = W_position.reshape(3 * H).astype(f32)
    out = _SC_CALL(*idxs, g0, g1, g2, g3, pos, wp)
    return out.reshape(B, L, H)


# speedup vs baseline: 5.5663x; 1.3730x over previous
import functools

import jax
import jax.numpy as jnp
from jax import lax
from jax.experimental import pallas as pl
from jax.experimental.pallas import tpu as pltpu
from jax.experimental.pallas import tpu_sc as plsc

B, L, H = 1024, 128, 128
BL = B * L
NF = 9
_GSIZES = (238, 192, 196, 144)
NG = len(_GSIZES)

NC, NS = 2, 16
NW = NC * NS
ROWS_PER_W = BL // NW
SB = 1024
NSB = ROWS_PER_W // SB
C = 64
NCH_SB = SB // C
HV = H // 16


def _make_sc_call():
    mesh = plsc.VectorSubcoreMesh(
        core_axis_name="c", subcore_axis_name="s", num_cores=NC, num_subcores=NS
    )
    scratch = (
        [pltpu.VMEM((n * H,), jnp.float32) for n in _GSIZES]
        + [pltpu.VMEM((3 * H,), jnp.float32)]
        + [pltpu.VMEM((NF, 8, L), jnp.int32)]
        + [pltpu.VMEM((8, 3 * L), jnp.float32)]
        + [pltpu.VMEM((C, H), jnp.float32) for _ in range(2)]
        + [pltpu.SemaphoreType.DMA for _ in range(2)]
    )

    @functools.partial(
        pl.kernel,
        mesh=mesh,
        out_type=jax.ShapeDtypeStruct((BL, H), jnp.float32),
        scratch_types=scratch,
    )
    def sc_kernel(i0, i1, i2, i3, i4, i5, i6, i7, i8,
                  tab0, tab1, tab2, tab3, pos_hbm, wp_hbm, out_hbm,
                  tv0, tv1, tv2, tv3, wp_v, ixv, pbv, ov0, ov1, so0, so1):
        idx_hbm = (i0, i1, i2, i3, i4, i5, i6, i7, i8)
        tab_v = (tv0, tv1, tv2, tv3)
        out_v = (ov0, ov1)
        sout = (so0, so1)

        wid = lax.axis_index("s") * NC + lax.axis_index("c")
        base0 = wid * ROWS_PER_W
        brow0 = wid * (ROWS_PER_W // L)

        for g, t in enumerate((tab0, tab1, tab2, tab3)):
            pltpu.sync_copy(t, tab_v[g])
        pltpu.sync_copy(wp_hbm, wp_v)

        def out_descr(chunk, slot):
            base = base0 + chunk * C
            return pltpu.make_async_copy(
                out_v[slot], out_hbm.at[pl.ds(base, C)], sout[slot])

        def compute_chunk(q, slot):
            wp_vecs = tuple(
                wp_v[pl.ds(k * H + j * 16, 16)] for k in range(3) for j in range(HV)
            )
            ov = out_v[slot]

            @plsc.parallel_loop(0, C // 16, carry=wp_vecs)
            def group_body(g, wp_c):
                off = q * C + g * 16
                row8 = off // L
                col = off % L
                iv = [ixv[f, row8, pl.ds(col, 16)] for f in range(NF)]
                cv = [
                    iv[0] * 2 + iv[5],
                    iv[1] * 12 + iv[2],
                    iv[3] * 14 + iv[4],
                    (iv[6] * 9 + iv[7]) * 2 + iv[8],
                ]
                prow = (off * 3) // (3 * L)
                pcol = (off * 3) % (3 * L)
                pvecs = [pbv[prow, pl.ds(pcol + m * 16, 16)] for m in range(3)]

                def extracts(rr):
                    idx = [cv[t][rr] for t in range(NG)]
                    pv = [
                        jnp.full((16,),
                                 pvecs[(rr * 3 + k) // 16][(rr * 3 + k) % 16],
                                 jnp.float32)
                        for k in range(3)
                    ]
                    return idx, pv

                def compute(pv, ld, j, row):
                    pacc = (pv[0] * wp_c[j] + pv[1] * wp_c[HV + j]
                            + pv[2] * wp_c[2 * HV + j])
                    ov[row, pl.ds(j * 16, 16)] = (
                        (ld[0] + ld[1]) + (ld[2] + ld[3])) + pacc

                DEPTH = 3
                cur = extracts(0)
                pending = []
                for rr in range(16):
                    idx_cur, pv_cur = cur
                    nxt = extracts(rr + 1) if rr < 15 else None
                    for j in range(HV):
                        ld = [tab_v[t][pl.ds(idx_cur[t] * H + j * 16, 16)]
                              for t in range(NG)]
                        pending.append((pv_cur, ld, j, g * 16 + rr))
                        if len(pending) > DEPTH:
                            compute(*pending.pop(0))
                    cur = nxt
                for p in pending:
                    compute(*p)
                return wp_c

        def sb_body(sb, carry):
            b8 = brow0 + sb * 8
            for f in range(NF):
                pltpu.sync_copy(idx_hbm[f].at[pl.ds(b8, 8)], ixv.at[f])
            pltpu.sync_copy(pos_hbm.at[pl.ds(b8, 8)], pbv)

            def q_body(q2, carry_q):
                for slot in range(2):
                    q = q2 * 2 + slot
                    chunk = sb * NCH_SB + q

                    @pl.when(chunk >= 2)
                    def _():
                        out_descr(chunk - 2, slot).wait()

                    compute_chunk(q, slot)
                    out_descr(chunk, slot).start()
                return carry_q

            lax.fori_loop(0, NCH_SB // 2, q_body, 0)
            return carry

        lax.fori_loop(0, NSB, sb_body, 0)
        out_descr(NSB * NCH_SB - 2, 0).wait()
        out_descr(NSB * NCH_SB - 1, 1).wait()

    return sc_kernel


_SC_CALL = _make_sc_call()


def kernel(atomic_number, formal_charge, degree, explicit_valence,
           implicit_valence, aromatic, hybridization, total_num_H, is_in_ring,
           W_atomic_number, W_formal_charge, W_degree, W_explicit_valence,
           W_implicit_valence, W_aromatic, W_hybridization, W_total_num_H,
           W_is_in_ring, position, W_position):
    idxs = [atomic_number, formal_charge, degree, explicit_valence,
            implicit_valence, aromatic, hybridization, total_num_H, is_in_ring]
    idxs = [i.astype(jnp.int32) for i in idxs]
    f32 = jnp.float32
    g0 = (W_atomic_number.astype(f32)[:, None, :]
          + W_aromatic.astype(f32)[None, :, :]).reshape(-1)
    g1 = (W_formal_charge.astype(f32)[:, None, :]
          + W_degree.astype(f32)[None, :, :]).reshape(-1)
    g2 = (W_explicit_valence.astype(f32)[:, None, :]
          + W_implicit_valence.astype(f32)[None, :, :]).reshape(-1)
    g3 = (W_hybridization.astype(f32)[:, None, None, :]
          + W_total_num_H.astype(f32)[None, :, None, :]
          + W_is_in_ring.astype(f32)[None, None, :, :]).reshape(-1)
    pos = position.astype(f32).reshape(B, 3 * L)
    wp = W_position.reshape(3 * H).astype(f32)
    out = _SC_CALL(*idxs, g0, g1, g2, g3, pos, wp)
    return out.reshape(B, L, H)


# speedup vs baseline: 6.3794x; 1.1461x over previous
import functools

import jax
import jax.numpy as jnp
from jax import lax
from jax.experimental import pallas as pl
from jax.experimental.pallas import tpu as pltpu
from jax.experimental.pallas import tpu_sc as plsc

B, L, H = 1024, 128, 128
BL = B * L
NF = 9
_GSIZES = (238, 192, 196, 144)
NG = len(_GSIZES)

NC, NS = 2, 16
NW = NC * NS
ROWS_PER_W = BL // NW
SB = 1024
NSB = ROWS_PER_W // SB
C = 64
NCH_SB = SB // C
HV = H // 16


def _make_sc_call():
    mesh = plsc.VectorSubcoreMesh(
        core_axis_name="c", subcore_axis_name="s", num_cores=NC, num_subcores=NS
    )
    scratch = (
        [pltpu.VMEM((n * H,), jnp.float32) for n in _GSIZES]
        + [pltpu.VMEM((3 * H,), jnp.float32)]
        + [pltpu.VMEM((NF, 8, L), jnp.int32)]
        + [pltpu.VMEM((8, 3 * L), jnp.float32)]
        + [pltpu.VMEM((C, H), jnp.float32) for _ in range(2)]
        + [pltpu.SemaphoreType.DMA for _ in range(3)]
    )

    @functools.partial(
        pl.kernel,
        mesh=mesh,
        out_type=jax.ShapeDtypeStruct((BL, H), jnp.float32),
        scratch_types=scratch,
    )
    def sc_kernel(i0, i1, i2, i3, i4, i5, i6, i7, i8,
                  tab0, tab1, tab2, tab3, pos_hbm, wp_hbm, out_hbm,
                  tv0, tv1, tv2, tv3, wp_v, ixv, pbv, ov0, ov1, so0, so1, sib):
        idx_hbm = (i0, i1, i2, i3, i4, i5, i6, i7, i8)
        tab_v = (tv0, tv1, tv2, tv3)
        out_v = (ov0, ov1)
        sout = (so0, so1)

        wid = lax.axis_index("s") * NC + lax.axis_index("c")
        base0 = wid * ROWS_PER_W
        brow0 = wid * (ROWS_PER_W // L)

        for g, t in enumerate((tab0, tab1, tab2, tab3)):
            pltpu.sync_copy(t, tab_v[g])
        pltpu.sync_copy(wp_hbm, wp_v)

        def out_descr(chunk, slot):
            base = base0 + chunk * C
            return pltpu.make_async_copy(
                out_v[slot], out_hbm.at[pl.ds(base, C)], sout[slot])

        def compute_chunk(q, slot):
            wp_vecs = tuple(
                wp_v[pl.ds(k * H + j * 16, 16)] for k in range(3) for j in range(HV)
            )
            ov = out_v[slot]

            @plsc.parallel_loop(0, C // 16, carry=wp_vecs)
            def group_body(g, wp_c):
                off = q * C + g * 16
                row8 = off // L
                col = off % L
                iv = [ixv[f, row8, pl.ds(col, 16)] for f in range(NF)]
                cv = [
                    iv[0] * 2 + iv[5],
                    iv[1] * 12 + iv[2],
                    iv[3] * 14 + iv[4],
                    (iv[6] * 9 + iv[7]) * 2 + iv[8],
                ]
                prow = (off * 3) // (3 * L)
                pcol = (off * 3) % (3 * L)
                pvecs = [pbv[prow, pl.ds(pcol + m * 16, 16)] for m in range(3)]

                def extracts(rr):
                    idx = [cv[t][rr] for t in range(NG)]
                    pv = [
                        jnp.full((16,),
                                 pvecs[(rr * 3 + k) // 16][(rr * 3 + k) % 16],
                                 jnp.float32)
                        for k in range(3)
                    ]
                    return idx, pv

                def compute(pv, ld, j, row):
                    pacc = (pv[0] * wp_c[j] + pv[1] * wp_c[HV + j]
                            + pv[2] * wp_c[2 * HV + j])
                    ov[row, pl.ds(j * 16, 16)] = (
                        (ld[0] + ld[1]) + (ld[2] + ld[3])) + pacc

                DEPTH = 3
                cur = extracts(0)
                pending = []
                for rr in range(16):
                    idx_cur, pv_cur = cur
                    nxt = extracts(rr + 1) if rr < 15 else None
                    for j in range(HV):
                        ld = [tab_v[t][pl.ds(idx_cur[t] * H + j * 16, 16)]
                              for t in range(NG)]
                        pending.append((pv_cur, ld, j, g * 16 + rr))
                        if len(pending) > DEPTH:
                            compute(*pending.pop(0))
                    cur = nxt
                for p in pending:
                    compute(*p)
                return wp_c

        def sb_body(sb, carry):
            b8 = brow0 + sb * 8
            ds = [pltpu.make_async_copy(idx_hbm[f].at[pl.ds(b8, 8)],
                                        ixv.at[f], sib) for f in range(NF)]
            ds.append(pltpu.make_async_copy(pos_hbm.at[pl.ds(b8, 8)], pbv, sib))
            for d in ds:
                d.start()
            for d in ds:
                d.wait()

            def q_body(q2, carry_q):
                for slot in range(2):
                    q = q2 * 2 + slot
                    chunk = sb * NCH_SB + q

                    @pl.when(chunk >= 2)
                    def _():
                        out_descr(chunk - 2, slot).wait()

                    compute_chunk(q, slot)
                    out_descr(chunk, slot).start()
                return carry_q

            lax.fori_loop(0, NCH_SB // 2, q_body, 0)
            return carry

        lax.fori_loop(0, NSB, sb_body, 0)
        out_descr(NSB * NCH_SB - 2, 0).wait()
        out_descr(NSB * NCH_SB - 1, 1).wait()

    return sc_kernel


_SC_CALL = _make_sc_call()


def kernel(atomic_number, formal_charge, degree, explicit_valence,
           implicit_valence, aromatic, hybridization, total_num_H, is_in_ring,
           W_atomic_number, W_formal_charge, W_degree, W_explicit_valence,
           W_implicit_valence, W_aromatic, W_hybridization, W_total_num_H,
           W_is_in_ring, position, W_position):
    idxs = [atomic_number, formal_charge, degree, explicit_valence,
            implicit_valence, aromatic, hybridization, total_num_H, is_in_ring]
    idxs = [i.astype(jnp.int32) for i in idxs]
    f32 = jnp.float32
    g0 = (W_atomic_number.astype(f32)[:, None, :]
          + W_aromatic.astype(f32)[None, :, :]).reshape(-1)
    g1 = (W_formal_charge.astype(f32)[:, None, :]
          + W_degree.astype(f32)[None, :, :]).reshape(-1)
    g2 = (W_explicit_valence.astype(f32)[:, None, :]
          + W_implicit_valence.astype(f32)[None, :, :]).reshape(-1)
    g3 = (W_hybridization.astype(f32)[:, None, None, :]
          + W_total_num_H.astype(f32)[None, :, None, :]
          + W_is_in_ring.astype(f32)[None, None, :, :]).reshape(-1)
    pos = position.astype(f32).reshape(B, 3 * L)
    wp = W_position.reshape(3 * H).astype(f32)
    out = _SC_CALL(*idxs, g0, g1, g2, g3, pos, wp)
    return out.reshape(B, L, H)
